# Initial kernel scaffold; baseline (speedup 1.0000x reference)
import jax
import jax.numpy as jnp
from jax.experimental import pallas as pl


def kernel(team, skill, embedding):
    raise NotImplementedError("write your pallas kernel here")


# speedup vs baseline: 1.5904x; 1.5904x over previous
import functools

import jax
import jax.numpy as jnp
from jax import lax
from jax.experimental import pallas as pl
from jax.experimental.pallas import tpu as pltpu
from jax.experimental.pallas import tpu_sc as plsc

N_HERO = 1000000
TEAM = 5
HID = 16
BATCH = 16384

NC = 2
NS = 16
NW = NC * NS
MPW = BATCH // NW
IPW = MPW * TEAM
CHUNK = 128
NCHUNK = IPW // CHUNK
NBLK = MPW // 16


def _fm_body(team_hbm, skill_hbm, emb_hbm, out_hbm, idx_v, rows_v, sk_v,
             out_v, sem_r, sem_s):
    w = lax.axis_index("s") * NC + lax.axis_index("c")
    base = w * MPW

    pltpu.sync_copy(team_hbm.at[pl.ds(base * TEAM, IPW)], idx_v)

    copies = []
    for j in range(NCHUNK):
        sl = pl.ds(j * CHUNK, CHUNK)
        copies.append(
            pltpu.async_copy(emb_hbm.at[idx_v.at[sl]], rows_v.at[sl, :], sem_r))
        copies.append(
            pltpu.async_copy(skill_hbm.at[idx_v.at[sl]], sk_v.at[sl], sem_s))
    for c in copies:
        c.wait()

    iota = lax.iota(jnp.int32, 16)
    iota5 = iota * TEAM
    z16 = jnp.zeros((16,), jnp.int32)
    zf = jnp.zeros((16,), jnp.float32)

    def blk(b, carry):
        rb = b * (16 * TEAM)
        rows = [iota5 + (rb + i) for i in range(TEAM)]
        tsk = zf
        for i in range(TEAM):
            tsk = tsk + plsc.load_gather(sk_v, [rows[i]])
        acc = zf
        for d in range(HID):
            cold = jnp.full((16,), d, jnp.int32)
            e = [plsc.load_gather(rows_v, [rows[i], cold]) for i in range(TEAM)]
            s = e[0] + e[1] + e[2] + e[3] + e[4]
            sq = e[0] * e[0] + e[1] * e[1] + e[2] * e[2] + e[3] * e[3] + e[4] * e[4]
            acc = acc + (s * s - sq)
        out_v[pl.ds(b * 16, 16)] = tsk + 0.5 * acc
        return carry

    lax.fori_loop(0, NBLK, blk, 0)
    pltpu.sync_copy(out_v, out_hbm.at[pl.ds(base, MPW)])


@jax.jit
def _fm(team_flat, skill, embedding):
    mesh = plsc.VectorSubcoreMesh(
        core_axis_name="c", subcore_axis_name="s", num_cores=NC,
        num_subcores=NS)
    f = pl.kernel(
        _fm_body,
        out_type=jax.ShapeDtypeStruct((BATCH,), jnp.float32),
        mesh=mesh,
        scratch_types=[
            pltpu.VMEM((IPW,), jnp.int32),
            pltpu.VMEM((IPW, HID), jnp.float32),
            pltpu.VMEM((IPW,), jnp.float32),
            pltpu.VMEM((MPW,), jnp.float32),
            pltpu.SemaphoreType.DMA,
            pltpu.SemaphoreType.DMA,
        ],
        compiler_params=pltpu.CompilerParams(
            needs_layout_passes=False, use_tc_tiling_on_sc=False),
    )
    return f(team_flat, skill, embedding)


def kernel(team, skill, embedding):
    team_flat = team.reshape(-1).astype(jnp.int32)
    out = _fm(team_flat, skill.reshape(-1), embedding)
    return out.reshape(-1, 1)
